# baseline (speedup 1.0000x reference)
import jax
import jax.numpy as jnp
from jax import lax
from jax.experimental import pallas as pl
from jax.experimental.pallas import tpu as pltpu
from jax.experimental.pallas import tpu_sc as plsc

_NC = 2
_NS = 16
_CHUNK = 128
_BN_SCALE = float(1.0 / (1.0 + 1e-5) ** 0.5)


def _sc_segment_sum(t, srcp, dstp, n_acc, with_count):
  nrows = t.shape[0] // _NC
  wh = t.shape[1]
  cpt = srcp.shape[0] // _NS
  rows_pt = n_acc // _NS
  zr = 32
  assert rows_pt % zr == 0 and cpt % 2 == 0

  nbuf = 4
  mesh = plsc.VectorSubcoreMesh(core_axis_name="c", subcore_axis_name="s",
                                num_cores=_NC, num_subcores=_NS)
  out_type = [jax.ShapeDtypeStruct((_NC * n_acc, wh), jnp.float32)]
  scratch = [
      pltpu.VMEM((cpt + nbuf, _CHUNK), jnp.int32),
      pltpu.VMEM((cpt, _CHUNK), jnp.int32),
      [pltpu.VMEM((_CHUNK, wh), jnp.float32) for _ in range(nbuf)],
      pltpu.VMEM((zr, wh), jnp.float32),
      pltpu.VMEM_SHARED((n_acc, wh), jnp.float32),
      [pltpu.SemaphoreType.DMA for _ in range(nbuf)],
      [pltpu.SemaphoreType.DMA for _ in range(nbuf)],
  ]
  if with_count:
    out_type.append(jax.ShapeDtypeStruct((_NC * n_acc, 16), jnp.float32))
    scratch += [
        pltpu.VMEM((_CHUNK, 16), jnp.float32),
        pltpu.VMEM((zr, 16), jnp.float32),
        pltpu.VMEM_SHARED((n_acc, 16), jnp.float32),
        pltpu.SemaphoreType.DMA,
    ]

  def body(t_hbm, src_hbm, dst_hbm, *refs):
    if with_count:
      (out_hbm, cnt_hbm, src_v, dst_v, bufs, zbuf, acc,
       gsem, ssem, ones_v, zbuf16, cacc, csem) = refs
    else:
      (out_hbm, src_v, dst_v, bufs, zbuf, acc, gsem, ssem) = refs
    c = lax.axis_index("c")
    s = lax.axis_index("s")
    tbl = t_hbm.at[pl.ds(pl.multiple_of(c * nrows, 8), nrows)]

    pltpu.sync_copy(src_hbm.at[pl.ds(s * cpt, cpt)], src_v.at[pl.ds(0, cpt)])
    pltpu.sync_copy(dst_hbm.at[pl.ds(s * cpt, cpt)], dst_v)
    z16i = jnp.zeros((16,), jnp.int32)
    z16f = jnp.zeros((16,), jnp.float32)
    for r in range(cpt, cpt + nbuf):
      for q in range(_CHUNK // 16):
        src_v[r, pl.ds(q * 16, 16)] = z16i
    for r in range(zr):
      for q in range(wh // 16):
        zbuf[r, pl.ds(q * 16, 16)] = z16f
    for r in range(rows_pt // zr):
      pltpu.sync_copy(zbuf, acc.at[pl.ds(s * rows_pt + r * zr, zr)])
    if with_count:
      o16 = jnp.ones((16,), jnp.float32)
      for r in range(_CHUNK):
        ones_v[r, pl.ds(0, 16)] = o16
      for r in range(zr):
        zbuf16[r, pl.ds(0, 16)] = z16f
      for r in range(rows_pt // zr):
        pltpu.sync_copy(zbuf16, cacc.at[pl.ds(s * rows_pt + r * zr, zr)])
    plsc.subcore_barrier()

    for b in range(nbuf):
      pltpu.async_copy(tbl.at[src_v.at[b]], bufs[b], gsem[b])

    def step(io, carry):
      jj = io * nbuf
      for b in range(nbuf):
        pltpu.make_async_copy(tbl.at[src_v.at[0]], bufs[b], gsem[b]).wait()
        pltpu.async_copy(bufs[b], acc.at[dst_v.at[jj + b]], ssem[b],
                         add=True)
        if with_count:
          @pl.when(c == (b % 2))
          def _():
            pltpu.async_copy(ones_v, cacc.at[dst_v.at[jj + b]], csem,
                             add=True)
      for b in range(nbuf):
        pltpu.make_async_copy(bufs[b], acc.at[dst_v.at[0]], ssem[b]).wait()
        pltpu.async_copy(tbl.at[src_v.at[jj + b + nbuf]], bufs[b], gsem[b])
      return carry

    lax.fori_loop(0, cpt // nbuf, step, 0)
    for b in range(nbuf):
      pltpu.make_async_copy(tbl.at[src_v.at[0]], bufs[b], gsem[b]).wait()
    if with_count:
      def cdrain(i, carry):
        pltpu.make_async_copy(ones_v, cacc.at[dst_v.at[0]], csem).wait()
        return carry
      lax.fori_loop(0, cpt // 2, cdrain, 0)
    plsc.subcore_barrier()

    off = pl.multiple_of(c * n_acc + s * rows_pt, 8)
    pltpu.sync_copy(acc.at[pl.ds(s * rows_pt, rows_pt)],
                    out_hbm.at[pl.ds(off, rows_pt)])
    if with_count:
      pltpu.sync_copy(cacc.at[pl.ds(s * rows_pt, rows_pt)],
                      cnt_hbm.at[pl.ds(off, rows_pt)])

  fn = pl.kernel(body, out_type=tuple(out_type), mesh=mesh,
                 scratch_types=tuple(scratch),
                 compiler_params=pltpu.CompilerParams(
                     use_tc_tiling_on_sc=False))
  return fn(t, srcp, dstp)


def _proj_body(x_ref, wa_ref, wb_ref, ylo_ref, yhi_ref, z_ref):
  xb = x_ref[...]
  y = jnp.dot(xb, wa_ref[...], preferred_element_type=jnp.float32)
  wh = y.shape[1] // 2
  ylo_ref[...] = y[:, :wh]
  yhi_ref[...] = y[:, wh:]
  z_ref[...] = jnp.dot(xb, wb_ref[...], preferred_element_type=jnp.float32)


def _project_split(x, wa_t, wb_t, bm):
  n, d = x.shape
  da, db = wa_t.shape[1], wb_t.shape[1]
  wh = da // 2
  return pl.pallas_call(
      _proj_body,
      grid=(n // bm,),
      in_specs=[pl.BlockSpec((bm, d), lambda i: (i, 0)),
                pl.BlockSpec((d, da), lambda i: (0, 0)),
                pl.BlockSpec((d, db), lambda i: (0, 0))],
      out_specs=[pl.BlockSpec((bm, wh), lambda i: (i, 0)),
                 pl.BlockSpec((bm, wh), lambda i: (i, 0)),
                 pl.BlockSpec((bm, db), lambda i: (i, 0))],
      out_shape=[jax.ShapeDtypeStruct((n, wh), jnp.float32),
                 jax.ShapeDtypeStruct((n, wh), jnp.float32),
                 jax.ShapeDtypeStruct((n, db), jnp.float32)],
  )(x, wa_t, wb_t)


def _combine_body(a_ref, c_ref, z_ref, bl_ref, g_ref, be_ref,
                  wa_ref, wb_ref, ylo_ref, yhi_ref, z2_ref):
  asum = jnp.concatenate([a_ref[0], a_ref[1]], axis=1)
  csum = c_ref[0][:, :1] + c_ref[1][:, :1]
  v = asum / jnp.maximum(csum, 1.0) + bl_ref[...] + z_ref[...]
  h = jnp.maximum(v * (g_ref[...] * _BN_SCALE) + be_ref[...], 0.0)
  y = jnp.dot(h, wa_ref[...], preferred_element_type=jnp.float32)
  wh = y.shape[1] // 2
  ylo_ref[...] = y[:, :wh]
  yhi_ref[...] = y[:, wh:]
  z2_ref[...] = jnp.dot(h, wb_ref[...], preferred_element_type=jnp.float32)


def _combine_project_split(a, cnt, z, bl, g, be, wa_t, wb_t, bm):
  n, w = z.shape
  ah = a.shape[2]
  da, db = wa_t.shape[1], wb_t.shape[1]
  wh = da // 2
  return pl.pallas_call(
      _combine_body,
      grid=(n // bm,),
      in_specs=[pl.BlockSpec((2, bm, ah), lambda i: (0, i, 0)),
                pl.BlockSpec((2, bm, 16), lambda i: (0, i, 0)),
                pl.BlockSpec((bm, w), lambda i: (i, 0)),
                pl.BlockSpec((1, w), lambda i: (0, 0)),
                pl.BlockSpec((1, w), lambda i: (0, 0)),
                pl.BlockSpec((1, w), lambda i: (0, 0)),
                pl.BlockSpec((w, da), lambda i: (0, 0)),
                pl.BlockSpec((w, db), lambda i: (0, 0))],
      out_specs=[pl.BlockSpec((bm, wh), lambda i: (i, 0)),
                 pl.BlockSpec((bm, wh), lambda i: (i, 0)),
                 pl.BlockSpec((bm, db), lambda i: (i, 0))],
      out_shape=[jax.ShapeDtypeStruct((n, wh), jnp.float32),
                 jax.ShapeDtypeStruct((n, wh), jnp.float32),
                 jax.ShapeDtypeStruct((n, db), jnp.float32)],
  )(a, cnt, z, bl, g, be, wa_t, wb_t)


def _final_body(a_ref, c_ref, z_ref, bl_ref, o_ref):
  asum = jnp.concatenate([a_ref[0], a_ref[1]], axis=1)
  csum = c_ref[0][:, :1] + c_ref[1][:, :1]
  o_ref[...] = asum / jnp.maximum(csum, 1.0) + bl_ref[...] + z_ref[...]


def _final_combine(a, cnt, z, bl, bm):
  n, w = z.shape
  ah = a.shape[2]
  return pl.pallas_call(
      _final_body,
      grid=(n // bm,),
      in_specs=[pl.BlockSpec((2, bm, ah), lambda i: (0, i, 0)),
                pl.BlockSpec((2, bm, 16), lambda i: (0, i, 0)),
                pl.BlockSpec((bm, w), lambda i: (i, 0)),
                pl.BlockSpec((1, w), lambda i: (0, 0))],
      out_specs=pl.BlockSpec((bm, w), lambda i: (i, 0)),
      out_shape=jax.ShapeDtypeStruct((n, w), jnp.float32),
  )(a, cnt, z, bl)


def kernel(x, edge_index, Wl1, bl1, Wr1, g1, be1, Wl2, bl2, Wr2, g2, be2,
           Wl3, bl3, Wr3):
  n = x.shape[0]
  e = edge_index.shape[1]
  bm = 1000
  n_acc = -(-(n + 1) // (_NS * 64)) * (_NS * 64)
  cpt = -(-e // (_NS * _CHUNK))
  cpt = -(-cpt // 8) * 8
  ep = _NS * cpt * _CHUNK

  src = edge_index[0]
  dst = edge_index[1]
  srcp = jnp.concatenate([src, jnp.zeros((ep - e,), jnp.int32)]
                         ).reshape(-1, _CHUNK)
  dstp = jnp.concatenate([dst, jnp.full((ep - e,), n, jnp.int32)]
                         ).reshape(-1, _CHUNK)

  def r2(v):
    return v.reshape(1, -1)

  def cat(lo, hi):
    return jnp.concatenate([lo, hi], axis=0)

  y1lo, y1hi, z1 = _project_split(x, Wl1.T, Wr1.T, bm)
  a1, cnt = _sc_segment_sum(cat(y1lo, y1hi), srcp, dstp, n_acc,
                            with_count=True)
  a1 = a1.reshape(_NC, n_acc, -1)
  cnt = cnt.reshape(_NC, n_acc, 16)
  y2lo, y2hi, z2 = _combine_project_split(a1, cnt, z1, r2(bl1), r2(g1),
                                          r2(be1), Wl2.T, Wr2.T, bm)
  a2 = _sc_segment_sum(cat(y2lo, y2hi), srcp, dstp, n_acc,
                       with_count=False)[0]
  a2 = a2.reshape(_NC, n_acc, -1)
  y3lo, y3hi, z3 = _combine_project_split(a2, cnt, z2, r2(bl2), r2(g2),
                                          r2(be2), Wl3.T, Wr3.T, bm)
  a3 = _sc_segment_sum(cat(y3lo, y3hi), srcp, dstp, n_acc,
                       with_count=False)[0]
  a3 = a3.reshape(_NC, n_acc, -1)
  return _final_combine(a3, cnt, z3, r2(bl3), bm)


# speedup vs baseline: 1.1835x; 1.1835x over previous
import jax
import jax.numpy as jnp
from jax import lax
from jax.experimental import pallas as pl
from jax.experimental.pallas import tpu as pltpu
from jax.experimental.pallas import tpu_sc as plsc

_NC = 2
_NS = 16
_CHUNK = 128
_BN_SCALE = float(1.0 / (1.0 + 1e-5) ** 0.5)


def _sc_segment_sum(t, srcp, dstp, n_acc, with_count):
  nrows = t.shape[0] // _NC
  wh = t.shape[1]
  cpt = srcp.shape[0] // _NS
  rows_pt = n_acc // _NS
  zr = 32
  assert rows_pt % zr == 0 and cpt % 2 == 0

  nbuf = 2
  mesh = plsc.VectorSubcoreMesh(core_axis_name="c", subcore_axis_name="s",
                                num_cores=_NC, num_subcores=_NS)
  out_type = [jax.ShapeDtypeStruct((_NC * n_acc, wh), jnp.float32)]
  scratch = [
      pltpu.VMEM((cpt + nbuf, _CHUNK), jnp.int32),
      pltpu.VMEM((cpt, _CHUNK), jnp.int32),
      [pltpu.VMEM((_CHUNK, wh), jnp.float32) for _ in range(nbuf)],
      pltpu.VMEM((zr, wh), jnp.float32),
      pltpu.VMEM_SHARED((n_acc, wh), jnp.float32),
      [pltpu.SemaphoreType.DMA for _ in range(nbuf)],
  ]
  if with_count:
    out_type.append(jax.ShapeDtypeStruct((_NC * n_acc, 16), jnp.float32))
    scratch += [
        pltpu.VMEM((_CHUNK, 16), jnp.float32),
        pltpu.VMEM((zr, 16), jnp.float32),
        pltpu.VMEM_SHARED((n_acc, 16), jnp.float32),
    ]

  def body(t_hbm, src_hbm, dst_hbm, *refs):
    if with_count:
      (out_hbm, cnt_hbm, src_v, dst_v, bufs, zbuf, acc,
       gsem, ones_v, zbuf16, cacc) = refs
    else:
      (out_hbm, src_v, dst_v, bufs, zbuf, acc, gsem) = refs
    c = lax.axis_index("c")
    s = lax.axis_index("s")
    tbl = t_hbm.at[pl.ds(pl.multiple_of(c * nrows, 8), nrows)]

    pltpu.sync_copy(src_hbm.at[pl.ds(s * cpt, cpt)], src_v.at[pl.ds(0, cpt)])
    pltpu.sync_copy(dst_hbm.at[pl.ds(s * cpt, cpt)], dst_v)
    z16i = jnp.zeros((16,), jnp.int32)
    z16f = jnp.zeros((16,), jnp.float32)
    for r in range(cpt, cpt + nbuf):
      for q in range(_CHUNK // 16):
        src_v[r, pl.ds(q * 16, 16)] = z16i
    for r in range(zr):
      for q in range(wh // 16):
        zbuf[r, pl.ds(q * 16, 16)] = z16f
    for r in range(rows_pt // zr):
      pltpu.sync_copy(zbuf, acc.at[pl.ds(s * rows_pt + r * zr, zr)])
    if with_count:
      o16 = jnp.ones((16,), jnp.float32)
      for r in range(_CHUNK):
        ones_v[r, pl.ds(0, 16)] = o16
      for r in range(zr):
        zbuf16[r, pl.ds(0, 16)] = z16f
      for r in range(rows_pt // zr):
        pltpu.sync_copy(zbuf16, cacc.at[pl.ds(s * rows_pt + r * zr, zr)])
    plsc.subcore_barrier()

    for b in range(nbuf):
      pltpu.async_copy(tbl.at[src_v.at[b]], bufs[b], gsem[b])

    def step(io, carry):
      jj = io * nbuf
      for b in range(nbuf):
        pltpu.make_async_copy(tbl.at[src_v.at[0]], bufs[b], gsem[b]).wait()
        pltpu.sync_copy(bufs[b], acc.at[dst_v.at[jj + b]], add=True)
        if with_count:
          @pl.when(c == b)
          def _():
            pltpu.sync_copy(ones_v, cacc.at[dst_v.at[jj + b]], add=True)
        pltpu.async_copy(tbl.at[src_v.at[jj + b + nbuf]], bufs[b], gsem[b])
      return carry

    lax.fori_loop(0, cpt // nbuf, step, 0)
    for b in range(nbuf):
      pltpu.make_async_copy(tbl.at[src_v.at[0]], bufs[b], gsem[b]).wait()
    plsc.subcore_barrier()

    off = pl.multiple_of(c * n_acc + s * rows_pt, 8)
    pltpu.sync_copy(acc.at[pl.ds(s * rows_pt, rows_pt)],
                    out_hbm.at[pl.ds(off, rows_pt)])
    if with_count:
      pltpu.sync_copy(cacc.at[pl.ds(s * rows_pt, rows_pt)],
                      cnt_hbm.at[pl.ds(off, rows_pt)])

  fn = pl.kernel(body, out_type=tuple(out_type), mesh=mesh,
                 scratch_types=tuple(scratch),
                 compiler_params=pltpu.CompilerParams(
                     use_tc_tiling_on_sc=False))
  return fn(t, srcp, dstp)


def _proj_body(x_ref, wa_ref, wb_ref, y_ref, z_ref):
  xb = x_ref[...]
  y = jnp.dot(xb, wa_ref[...], preferred_element_type=jnp.float32)
  wh = y.shape[1] // 2
  y_ref[0] = y[:, :wh]
  y_ref[1] = y[:, wh:]
  z_ref[...] = jnp.dot(xb, wb_ref[...], preferred_element_type=jnp.float32)


def _project_split(x, wa_t, wb_t, bm):
  n, d = x.shape
  da, db = wa_t.shape[1], wb_t.shape[1]
  wh = da // 2
  return pl.pallas_call(
      _proj_body,
      grid=(n // bm,),
      in_specs=[pl.BlockSpec((bm, d), lambda i: (i, 0)),
                pl.BlockSpec((d, da), lambda i: (0, 0)),
                pl.BlockSpec((d, db), lambda i: (0, 0))],
      out_specs=[pl.BlockSpec((2, bm, wh), lambda i: (0, i, 0)),
                 pl.BlockSpec((bm, db), lambda i: (i, 0))],
      out_shape=[jax.ShapeDtypeStruct((2, n, wh), jnp.float32),
                 jax.ShapeDtypeStruct((n, db), jnp.float32)],
  )(x, wa_t, wb_t)


def _combine_body(a_ref, c_ref, z_ref, bl_ref, g_ref, be_ref,
                  wa_ref, wb_ref, y_ref, z2_ref):
  asum = jnp.concatenate([a_ref[0], a_ref[1]], axis=1)
  csum = c_ref[0][:, :1] + c_ref[1][:, :1]
  v = asum / jnp.maximum(csum, 1.0) + bl_ref[...] + z_ref[...]
  h = jnp.maximum(v * (g_ref[...] * _BN_SCALE) + be_ref[...], 0.0)
  y = jnp.dot(h, wa_ref[...], preferred_element_type=jnp.float32)
  wh = y.shape[1] // 2
  y_ref[0] = y[:, :wh]
  y_ref[1] = y[:, wh:]
  z2_ref[...] = jnp.dot(h, wb_ref[...], preferred_element_type=jnp.float32)


def _combine_project_split(a, cnt, z, bl, g, be, wa_t, wb_t, bm):
  n, w = z.shape
  ah = a.shape[2]
  da, db = wa_t.shape[1], wb_t.shape[1]
  wh = da // 2
  return pl.pallas_call(
      _combine_body,
      grid=(n // bm,),
      in_specs=[pl.BlockSpec((2, bm, ah), lambda i: (0, i, 0)),
                pl.BlockSpec((2, bm, 16), lambda i: (0, i, 0)),
                pl.BlockSpec((bm, w), lambda i: (i, 0)),
                pl.BlockSpec((1, w), lambda i: (0, 0)),
                pl.BlockSpec((1, w), lambda i: (0, 0)),
                pl.BlockSpec((1, w), lambda i: (0, 0)),
                pl.BlockSpec((w, da), lambda i: (0, 0)),
                pl.BlockSpec((w, db), lambda i: (0, 0))],
      out_specs=[pl.BlockSpec((2, bm, wh), lambda i: (0, i, 0)),
                 pl.BlockSpec((bm, db), lambda i: (i, 0))],
      out_shape=[jax.ShapeDtypeStruct((2, n, wh), jnp.float32),
                 jax.ShapeDtypeStruct((n, db), jnp.float32)],
  )(a, cnt, z, bl, g, be, wa_t, wb_t)


def _final_body(a_ref, c_ref, z_ref, bl_ref, o_ref):
  asum = jnp.concatenate([a_ref[0], a_ref[1]], axis=1)
  csum = c_ref[0][:, :1] + c_ref[1][:, :1]
  o_ref[...] = asum / jnp.maximum(csum, 1.0) + bl_ref[...] + z_ref[...]


def _final_combine(a, cnt, z, bl, bm):
  n, w = z.shape
  ah = a.shape[2]
  return pl.pallas_call(
      _final_body,
      grid=(n // bm,),
      in_specs=[pl.BlockSpec((2, bm, ah), lambda i: (0, i, 0)),
                pl.BlockSpec((2, bm, 16), lambda i: (0, i, 0)),
                pl.BlockSpec((bm, w), lambda i: (i, 0)),
                pl.BlockSpec((1, w), lambda i: (0, 0))],
      out_specs=pl.BlockSpec((bm, w), lambda i: (i, 0)),
      out_shape=jax.ShapeDtypeStruct((n, w), jnp.float32),
  )(a, cnt, z, bl)


def kernel(x, edge_index, Wl1, bl1, Wr1, g1, be1, Wl2, bl2, Wr2, g2, be2,
           Wl3, bl3, Wr3):
  n = x.shape[0]
  e = edge_index.shape[1]
  bm = 1000
  n_acc = -(-(n + 1) // (_NS * 64)) * (_NS * 64)
  cpt = -(-e // (_NS * _CHUNK))
  cpt = -(-cpt // 8) * 8
  ep = _NS * cpt * _CHUNK

  src = edge_index[0]
  dst = edge_index[1]
  srcp = jnp.concatenate([src, jnp.zeros((ep - e,), jnp.int32)]
                         ).reshape(-1, _CHUNK)
  dstp = jnp.concatenate([dst, jnp.full((ep - e,), n, jnp.int32)]
                         ).reshape(-1, _CHUNK)

  def r2(v):
    return v.reshape(1, -1)

  def flat(y):
    return y.reshape(-1, y.shape[2])

  y1, z1 = _project_split(x, Wl1.T, Wr1.T, bm)
  a1, cnt = _sc_segment_sum(flat(y1), srcp, dstp, n_acc, with_count=True)
  a1 = a1.reshape(_NC, n_acc, -1)
  cnt = cnt.reshape(_NC, n_acc, 16)
  y2, z2 = _combine_project_split(a1, cnt, z1, r2(bl1), r2(g1),
                                  r2(be1), Wl2.T, Wr2.T, bm)
  a2 = _sc_segment_sum(flat(y2), srcp, dstp, n_acc, with_count=False)[0]
  a2 = a2.reshape(_NC, n_acc, -1)
  y3, z3 = _combine_project_split(a2, cnt, z2, r2(bl2), r2(g2),
                                  r2(be2), Wl3.T, Wr3.T, bm)
  a3 = _sc_segment_sum(flat(y3), srcp, dstp, n_acc, with_count=False)[0]
  a3 = a3.reshape(_NC, n_acc, -1)
  return _final_combine(a3, cnt, z3, r2(bl3), bm)


# speedup vs baseline: 2.4412x; 2.0627x over previous
import jax
import jax.numpy as jnp
from jax import lax
from jax.experimental import pallas as pl
from jax.experimental.pallas import tpu as pltpu
from jax.experimental.pallas import tpu_sc as plsc

_NC = 2
_NS = 16
_CHUNK = 128
_BN_SCALE = float(1.0 / (1.0 + 1e-5) ** 0.5)


def _unpack_chunk(packed_v, src_u, dst_u, j, slot):
  m16 = jnp.full((16,), 0xFFFF, jnp.int32)
  for q in range(_CHUNK // 16):
    v = packed_v[j, pl.ds(q * 16, 16)]
    src_u[slot, pl.ds(q * 16, 16)] = v & m16
    dst_u[slot, pl.ds(q * 16, 16)] = v >> 16


def _sc_segment_sum(t, packed, n_acc):
  wh = t.shape[1]
  cpt = (packed.shape[0] - 8) // _NS
  rows_pt = n_acc // _NS
  zr = 32
  assert rows_pt % zr == 0 and cpt % 2 == 0

  mesh = plsc.VectorSubcoreMesh(core_axis_name="c", subcore_axis_name="s",
                                num_cores=_NC, num_subcores=_NS)
  scratch = [
      pltpu.VMEM((cpt + 4, _CHUNK), jnp.int32),
      pltpu.VMEM((4, _CHUNK), jnp.int32),
      pltpu.VMEM((4, _CHUNK), jnp.int32),
      [pltpu.VMEM((_CHUNK, wh), jnp.float32) for _ in range(2)],
      pltpu.VMEM((zr, wh), jnp.float32),
      pltpu.VMEM_SHARED((n_acc, wh), jnp.float32),
      pltpu.VMEM_SHARED((n_acc, wh), jnp.float32),
      [pltpu.SemaphoreType.DMA for _ in range(2)],
  ]

  def body(t_hbm, packed_hbm, out_hbm, packed_v, src_u, dst_u, bufs, zbuf,
           tbl, acc, gsem):
    c = lax.axis_index("c")
    s = lax.axis_index("s")

    pltpu.sync_copy(packed_hbm.at[pl.ds(s * cpt, cpt + 4)], packed_v)
    toff = pl.multiple_of(c * n_acc + s * rows_pt, 8)
    pltpu.sync_copy(t_hbm.at[pl.ds(toff, rows_pt)],
                    tbl.at[pl.ds(s * rows_pt, rows_pt)])
    z16f = jnp.zeros((16,), jnp.float32)
    for r in range(zr):
      for q in range(wh // 16):
        zbuf[r, pl.ds(q * 16, 16)] = z16f
    for r in range(rows_pt // zr):
      pltpu.sync_copy(zbuf, acc.at[pl.ds(s * rows_pt + r * zr, zr)])
    for j in range(4):
      _unpack_chunk(packed_v, src_u, dst_u, j, j)
    plsc.subcore_barrier()

    pltpu.async_copy(tbl.at[src_u.at[0]], bufs[0], gsem[0])
    pltpu.async_copy(tbl.at[src_u.at[1]], bufs[1], gsem[1])

    def step(io, carry):
      jj = io * 2
      for b in range(2):
        j = jj + b
        pltpu.make_async_copy(tbl.at[src_u.at[0]], bufs[b], gsem[b]).wait()
        pltpu.sync_copy(bufs[b], acc.at[dst_u.at[lax.rem(j, 4)]], add=True)
        pltpu.async_copy(tbl.at[src_u.at[lax.rem(j + 2, 4)]], bufs[b],
                         gsem[b])
        _unpack_chunk(packed_v, src_u, dst_u, j + 4, lax.rem(j, 4))
      return carry

    lax.fori_loop(0, cpt // 2, step, 0)
    for b in range(2):
      pltpu.make_async_copy(tbl.at[src_u.at[0]], bufs[b], gsem[b]).wait()
    plsc.subcore_barrier()

    pltpu.sync_copy(acc.at[pl.ds(s * rows_pt, rows_pt)],
                    out_hbm.at[pl.ds(toff, rows_pt)])

  fn = pl.kernel(body,
                 out_type=jax.ShapeDtypeStruct((_NC * n_acc, wh),
                                               jnp.float32),
                 mesh=mesh, scratch_types=tuple(scratch),
                 compiler_params=pltpu.CompilerParams(
                     use_tc_tiling_on_sc=False))
  return fn(t, packed)


def _sc_degree_count(packed, n_acc):
  cpt = (packed.shape[0] - 8) // _NS
  rows_pt = n_acc // _NS
  zr = 32
  mesh = plsc.VectorSubcoreMesh(core_axis_name="c", subcore_axis_name="s",
                                num_cores=_NC, num_subcores=_NS)
  scratch = [
      pltpu.VMEM((cpt, _CHUNK), jnp.int32),
      pltpu.VMEM((1, _CHUNK), jnp.int32),
      pltpu.VMEM((1, _CHUNK), jnp.int32),
      pltpu.VMEM((_CHUNK, 16), jnp.float32),
      pltpu.VMEM((zr, 16), jnp.float32),
      pltpu.VMEM_SHARED((n_acc, 16), jnp.float32),
  ]

  def body(packed_hbm, cnt_hbm, packed_v, src_u, dst_u, ones_v, zbuf16,
           cacc):
    c = lax.axis_index("c")
    s = lax.axis_index("s")
    pltpu.sync_copy(packed_hbm.at[pl.ds(s * cpt, cpt)], packed_v)
    z16f = jnp.zeros((16,), jnp.float32)
    o16 = jnp.ones((16,), jnp.float32)
    for r in range(_CHUNK):
      ones_v[r, pl.ds(0, 16)] = o16
    for r in range(zr):
      zbuf16[r, pl.ds(0, 16)] = z16f
    for r in range(rows_pt // zr):
      pltpu.sync_copy(zbuf16, cacc.at[pl.ds(s * rows_pt + r * zr, zr)])
    plsc.subcore_barrier()

    def step(j, carry):
      @pl.when(lax.rem(j, 2) == c)
      def _():
        _unpack_chunk(packed_v, src_u, dst_u, j, 0)
        pltpu.sync_copy(ones_v, cacc.at[dst_u.at[0]], add=True)
      return carry

    lax.fori_loop(0, cpt, step, 0)
    plsc.subcore_barrier()
    off = pl.multiple_of(c * n_acc + s * rows_pt, 8)
    pltpu.sync_copy(cacc.at[pl.ds(s * rows_pt, rows_pt)],
                    cnt_hbm.at[pl.ds(off, rows_pt)])

  fn = pl.kernel(body,
                 out_type=jax.ShapeDtypeStruct((_NC * n_acc, 16),
                                               jnp.float32),
                 mesh=mesh, scratch_types=tuple(scratch),
                 compiler_params=pltpu.CompilerParams(
                     use_tc_tiling_on_sc=False))
  return fn(packed)


def _proj_body(x_ref, wa_ref, wb_ref, y_ref, z_ref):
  xb = x_ref[...]
  y = jnp.dot(xb, wa_ref[...], preferred_element_type=jnp.float32)
  wh = y.shape[1] // 2
  y_ref[0] = y[:, :wh]
  y_ref[1] = y[:, wh:]
  z_ref[...] = jnp.dot(xb, wb_ref[...], preferred_element_type=jnp.float32)


def _project_split(x, wa_t, wb_t, bm, n_acc):
  n, d = x.shape
  da, db = wa_t.shape[1], wb_t.shape[1]
  wh = da // 2
  return pl.pallas_call(
      _proj_body,
      grid=(n // bm,),
      in_specs=[pl.BlockSpec((bm, d), lambda i: (i, 0)),
                pl.BlockSpec((d, da), lambda i: (0, 0)),
                pl.BlockSpec((d, db), lambda i: (0, 0))],
      out_specs=[pl.BlockSpec((2, bm, wh), lambda i: (0, i, 0)),
                 pl.BlockSpec((bm, db), lambda i: (i, 0))],
      out_shape=[jax.ShapeDtypeStruct((2, n_acc, wh), jnp.float32),
                 jax.ShapeDtypeStruct((n, db), jnp.float32)],
  )(x, wa_t, wb_t)


def _combine_body(a_ref, c_ref, z_ref, bl_ref, g_ref, be_ref,
                  wa_ref, wb_ref, y_ref, z2_ref):
  asum = jnp.concatenate([a_ref[0], a_ref[1]], axis=1)
  csum = c_ref[0][:, :1] + c_ref[1][:, :1]
  v = asum / jnp.maximum(csum, 1.0) + bl_ref[...] + z_ref[...]
  h = jnp.maximum(v * (g_ref[...] * _BN_SCALE) + be_ref[...], 0.0)
  y = jnp.dot(h, wa_ref[...], preferred_element_type=jnp.float32)
  wh = y.shape[1] // 2
  y_ref[0] = y[:, :wh]
  y_ref[1] = y[:, wh:]
  z2_ref[...] = jnp.dot(h, wb_ref[...], preferred_element_type=jnp.float32)


def _combine_project_split(a, cnt, z, bl, g, be, wa_t, wb_t, bm, n_acc):
  n, w = z.shape
  ah = a.shape[2]
  da, db = wa_t.shape[1], wb_t.shape[1]
  wh = da // 2
  return pl.pallas_call(
      _combine_body,
      grid=(n // bm,),
      in_specs=[pl.BlockSpec((2, bm, ah), lambda i: (0, i, 0)),
                pl.BlockSpec((2, bm, 16), lambda i: (0, i, 0)),
                pl.BlockSpec((bm, w), lambda i: (i, 0)),
                pl.BlockSpec((1, w), lambda i: (0, 0)),
                pl.BlockSpec((1, w), lambda i: (0, 0)),
                pl.BlockSpec((1, w), lambda i: (0, 0)),
                pl.BlockSpec((w, da), lambda i: (0, 0)),
                pl.BlockSpec((w, db), lambda i: (0, 0))],
      out_specs=[pl.BlockSpec((2, bm, wh), lambda i: (0, i, 0)),
                 pl.BlockSpec((bm, db), lambda i: (i, 0))],
      out_shape=[jax.ShapeDtypeStruct((2, n_acc, wh), jnp.float32),
                 jax.ShapeDtypeStruct((n, db), jnp.float32)],
  )(a, cnt, z, bl, g, be, wa_t, wb_t)


def _final_body(a_ref, c_ref, z_ref, bl_ref, o_ref):
  asum = jnp.concatenate([a_ref[0], a_ref[1]], axis=1)
  csum = c_ref[0][:, :1] + c_ref[1][:, :1]
  o_ref[...] = asum / jnp.maximum(csum, 1.0) + bl_ref[...] + z_ref[...]


def _final_combine(a, cnt, z, bl, bm):
  n, w = z.shape
  ah = a.shape[2]
  return pl.pallas_call(
      _final_body,
      grid=(n // bm,),
      in_specs=[pl.BlockSpec((2, bm, ah), lambda i: (0, i, 0)),
                pl.BlockSpec((2, bm, 16), lambda i: (0, i, 0)),
                pl.BlockSpec((bm, w), lambda i: (i, 0)),
                pl.BlockSpec((1, w), lambda i: (0, 0))],
      out_specs=pl.BlockSpec((bm, w), lambda i: (i, 0)),
      out_shape=jax.ShapeDtypeStruct((n, w), jnp.float32),
  )(a, cnt, z, bl)


def kernel(x, edge_index, Wl1, bl1, Wr1, g1, be1, Wl2, bl2, Wr2, g2, be2,
           Wl3, bl3, Wr3):
  n = x.shape[0]
  e = edge_index.shape[1]
  bm = 1000
  n_acc = -(-(n + 1) // (_NS * 64)) * (_NS * 64)
  cpt = -(-e // (_NS * _CHUNK))
  cpt = -(-cpt // 8) * 8
  ep = _NS * cpt * _CHUNK

  src = edge_index[0]
  dst = edge_index[1]
  packed = jnp.concatenate([
      src + dst * 65536,
      jnp.full((ep - e,), n * 65536, jnp.int32),
      jnp.zeros((8 * _CHUNK,), jnp.int32),
  ]).reshape(-1, _CHUNK)

  def r2(v):
    return v.reshape(1, -1)

  def flat(y):
    return y.reshape(-1, y.shape[2])

  cnt = _sc_degree_count(packed, n_acc).reshape(_NC, n_acc, 16)
  y1, z1 = _project_split(x, Wl1.T, Wr1.T, bm, n_acc)
  a1 = _sc_segment_sum(flat(y1), packed, n_acc).reshape(_NC, n_acc, -1)
  y2, z2 = _combine_project_split(a1, cnt, z1, r2(bl1), r2(g1),
                                  r2(be1), Wl2.T, Wr2.T, bm, n_acc)
  a2 = _sc_segment_sum(flat(y2), packed, n_acc).reshape(_NC, n_acc, -1)
  y3, z3 = _combine_project_split(a2, cnt, z2, r2(bl2), r2(g2),
                                  r2(be2), Wl3.T, Wr3.T, bm, n_acc)
  a3 = _sc_segment_sum(flat(y3), packed, n_acc).reshape(_NC, n_acc, -1)
  return _final_combine(a3, cnt, z3, r2(bl3), bm)


# speedup vs baseline: 2.5428x; 1.0416x over previous
import jax
import jax.numpy as jnp
from jax import lax
from jax.experimental import pallas as pl
from jax.experimental.pallas import tpu as pltpu
from jax.experimental.pallas import tpu_sc as plsc

_NC = 2
_NS = 16
_CHUNK = 128
_BN_SCALE = float(1.0 / (1.0 + 1e-5) ** 0.5)


def _unpack_chunk(packed_v, src_u, dst_u, j, slot):
  m16 = jnp.full((16,), 0xFFFF, jnp.int32)
  for q in range(_CHUNK // 16):
    v = packed_v[j, pl.ds(q * 16, 16)]
    src_u[slot, pl.ds(q * 16, 16)] = v & m16
    dst_u[slot, pl.ds(q * 16, 16)] = v >> 16


def _sc_segment_sum(t, packed, n_acc):
  wh = t.shape[1]
  cpt = (packed.shape[0] - 8) // _NS
  rows_pt = n_acc // _NS
  zr = 8
  assert rows_pt % zr == 0 and cpt % 3 == 0

  mesh = plsc.VectorSubcoreMesh(core_axis_name="c", subcore_axis_name="s",
                                num_cores=_NC, num_subcores=_NS)
  scratch = [
      pltpu.VMEM((cpt, _CHUNK), jnp.int32),
      pltpu.VMEM((8, _CHUNK), jnp.int32),
      pltpu.VMEM((8, _CHUNK), jnp.int32),
      [pltpu.VMEM((_CHUNK, wh), jnp.float32) for _ in range(3)],
      pltpu.VMEM((zr, wh), jnp.float32),
      pltpu.VMEM_SHARED((n_acc, wh), jnp.float32),
      pltpu.VMEM_SHARED((n_acc, wh), jnp.float32),
      [pltpu.SemaphoreType.DMA for _ in range(3)],
      [pltpu.SemaphoreType.DMA for _ in range(3)],
  ]

  def body(t_hbm, packed_hbm, out_hbm, packed_v, src_u, dst_u, bufs, zbuf,
           tbl, acc, gsem, ssem):
    c = lax.axis_index("c")
    s = lax.axis_index("s")

    pltpu.sync_copy(packed_hbm.at[pl.ds(s * cpt, cpt)], packed_v)
    toff = pl.multiple_of(c * n_acc + s * rows_pt, 8)
    pltpu.sync_copy(t_hbm.at[pl.ds(toff, rows_pt)],
                    tbl.at[pl.ds(s * rows_pt, rows_pt)])
    z16f = jnp.zeros((16,), jnp.float32)
    for r in range(zr):
      for q in range(wh // 16):
        zbuf[r, pl.ds(q * 16, 16)] = z16f
    for r in range(rows_pt // zr):
      pltpu.sync_copy(zbuf, acc.at[pl.ds(s * rows_pt + r * zr, zr)])
    for j in range(8):
      _unpack_chunk(packed_v, src_u, dst_u, j, j)
    plsc.subcore_barrier()

    def gat(j, b):
      pltpu.async_copy(tbl.at[src_u.at[lax.rem(j, 8)]], bufs[b], gsem[b])

    def gwait(b):
      pltpu.make_async_copy(tbl.at[src_u.at[0]], bufs[b], gsem[b]).wait()

    def scat(j, b):
      pltpu.async_copy(bufs[b], acc.at[dst_u.at[lax.rem(j, 8)]], ssem[b],
                       add=True)

    def swait(b):
      pltpu.make_async_copy(bufs[b], acc.at[dst_u.at[0]], ssem[b]).wait()

    def slot(j, b, first):
      gwait(b)
      scat(j, b)
      if not first:
        swait((b + 2) % 3)
      gat(j + 2, (b + 2) % 3)
      _unpack_chunk(packed_v, src_u, dst_u, jnp.minimum(j + 7, cpt - 1),
                    lax.rem(j + 7, 8))

    gat(0, 0)
    gat(1, 1)
    for b in range(3):
      slot(b, b, first=(b == 0))

    def step(io, carry):
      jj = io * 3
      for b in range(3):
        slot(jj + b, b, first=False)
      return carry

    lax.fori_loop(1, cpt // 3, step, 0)
    gwait(0)
    gwait(1)
    swait((cpt + 2) % 3)
    plsc.subcore_barrier()

    pltpu.sync_copy(acc.at[pl.ds(s * rows_pt, rows_pt)],
                    out_hbm.at[pl.ds(toff, rows_pt)])

  fn = pl.kernel(body,
                 out_type=jax.ShapeDtypeStruct((_NC * n_acc, wh),
                                               jnp.float32),
                 mesh=mesh, scratch_types=tuple(scratch),
                 compiler_params=pltpu.CompilerParams(
                     use_tc_tiling_on_sc=False))
  return fn(t, packed)


def _sc_degree_count(packed, n_acc):
  cpt = (packed.shape[0] - 8) // _NS
  rows_pt = n_acc // _NS
  zr = 32
  mesh = plsc.VectorSubcoreMesh(core_axis_name="c", subcore_axis_name="s",
                                num_cores=_NC, num_subcores=_NS)
  scratch = [
      pltpu.VMEM((cpt, _CHUNK), jnp.int32),
      pltpu.VMEM((1, _CHUNK), jnp.int32),
      pltpu.VMEM((1, _CHUNK), jnp.int32),
      pltpu.VMEM((_CHUNK, 16), jnp.float32),
      pltpu.VMEM((zr, 16), jnp.float32),
      pltpu.VMEM_SHARED((n_acc, 16), jnp.float32),
  ]

  def body(packed_hbm, cnt_hbm, packed_v, src_u, dst_u, ones_v, zbuf16,
           cacc):
    c = lax.axis_index("c")
    s = lax.axis_index("s")
    pltpu.sync_copy(packed_hbm.at[pl.ds(s * cpt, cpt)], packed_v)
    z16f = jnp.zeros((16,), jnp.float32)
    o16 = jnp.ones((16,), jnp.float32)
    for r in range(_CHUNK):
      ones_v[r, pl.ds(0, 16)] = o16
    for r in range(zr):
      zbuf16[r, pl.ds(0, 16)] = z16f
    for r in range(rows_pt // zr):
      pltpu.sync_copy(zbuf16, cacc.at[pl.ds(s * rows_pt + r * zr, zr)])
    plsc.subcore_barrier()

    def step(j, carry):
      @pl.when(lax.rem(j, 2) == c)
      def _():
        _unpack_chunk(packed_v, src_u, dst_u, j, 0)
        pltpu.sync_copy(ones_v, cacc.at[dst_u.at[0]], add=True)
      return carry

    lax.fori_loop(0, cpt, step, 0)
    plsc.subcore_barrier()
    off = pl.multiple_of(c * n_acc + s * rows_pt, 8)
    pltpu.sync_copy(cacc.at[pl.ds(s * rows_pt, rows_pt)],
                    cnt_hbm.at[pl.ds(off, rows_pt)])

  fn = pl.kernel(body,
                 out_type=jax.ShapeDtypeStruct((_NC * n_acc, 16),
                                               jnp.float32),
                 mesh=mesh, scratch_types=tuple(scratch),
                 compiler_params=pltpu.CompilerParams(
                     use_tc_tiling_on_sc=False))
  return fn(packed)


def _proj_body(x_ref, wa_ref, wb_ref, y_ref, z_ref):
  xb = x_ref[...]
  y = jnp.dot(xb, wa_ref[...], preferred_element_type=jnp.float32)
  wh = y.shape[1] // 2
  y_ref[0] = y[:, :wh]
  y_ref[1] = y[:, wh:]
  z_ref[...] = jnp.dot(xb, wb_ref[...], preferred_element_type=jnp.float32)


def _project_split(x, wa_t, wb_t, bm, n_acc):
  n, d = x.shape
  da, db = wa_t.shape[1], wb_t.shape[1]
  wh = da // 2
  return pl.pallas_call(
      _proj_body,
      grid=(n // bm,),
      in_specs=[pl.BlockSpec((bm, d), lambda i: (i, 0)),
                pl.BlockSpec((d, da), lambda i: (0, 0)),
                pl.BlockSpec((d, db), lambda i: (0, 0))],
      out_specs=[pl.BlockSpec((2, bm, wh), lambda i: (0, i, 0)),
                 pl.BlockSpec((bm, db), lambda i: (i, 0))],
      out_shape=[jax.ShapeDtypeStruct((2, n_acc, wh), jnp.float32),
                 jax.ShapeDtypeStruct((n, db), jnp.float32)],
  )(x, wa_t, wb_t)


def _combine_body(a_ref, c_ref, z_ref, bl_ref, g_ref, be_ref,
                  wa_ref, wb_ref, y_ref, z2_ref):
  asum = jnp.concatenate([a_ref[0], a_ref[1]], axis=1)
  csum = c_ref[0][:, :1] + c_ref[1][:, :1]
  v = asum / jnp.maximum(csum, 1.0) + bl_ref[...] + z_ref[...]
  h = jnp.maximum(v * (g_ref[...] * _BN_SCALE) + be_ref[...], 0.0)
  y = jnp.dot(h, wa_ref[...], preferred_element_type=jnp.float32)
  wh = y.shape[1] // 2
  y_ref[0] = y[:, :wh]
  y_ref[1] = y[:, wh:]
  z2_ref[...] = jnp.dot(h, wb_ref[...], preferred_element_type=jnp.float32)


def _combine_project_split(a, cnt, z, bl, g, be, wa_t, wb_t, bm, n_acc):
  n, w = z.shape
  ah = a.shape[2]
  da, db = wa_t.shape[1], wb_t.shape[1]
  wh = da // 2
  return pl.pallas_call(
      _combine_body,
      grid=(n // bm,),
      in_specs=[pl.BlockSpec((2, bm, ah), lambda i: (0, i, 0)),
                pl.BlockSpec((2, bm, 16), lambda i: (0, i, 0)),
                pl.BlockSpec((bm, w), lambda i: (i, 0)),
                pl.BlockSpec((1, w), lambda i: (0, 0)),
                pl.BlockSpec((1, w), lambda i: (0, 0)),
                pl.BlockSpec((1, w), lambda i: (0, 0)),
                pl.BlockSpec((w, da), lambda i: (0, 0)),
                pl.BlockSpec((w, db), lambda i: (0, 0))],
      out_specs=[pl.BlockSpec((2, bm, wh), lambda i: (0, i, 0)),
                 pl.BlockSpec((bm, db), lambda i: (i, 0))],
      out_shape=[jax.ShapeDtypeStruct((2, n_acc, wh), jnp.float32),
                 jax.ShapeDtypeStruct((n, db), jnp.float32)],
  )(a, cnt, z, bl, g, be, wa_t, wb_t)


def _final_body(a_ref, c_ref, z_ref, bl_ref, o_ref):
  asum = jnp.concatenate([a_ref[0], a_ref[1]], axis=1)
  csum = c_ref[0][:, :1] + c_ref[1][:, :1]
  o_ref[...] = asum / jnp.maximum(csum, 1.0) + bl_ref[...] + z_ref[...]


def _final_combine(a, cnt, z, bl, bm):
  n, w = z.shape
  ah = a.shape[2]
  return pl.pallas_call(
      _final_body,
      grid=(n // bm,),
      in_specs=[pl.BlockSpec((2, bm, ah), lambda i: (0, i, 0)),
                pl.BlockSpec((2, bm, 16), lambda i: (0, i, 0)),
                pl.BlockSpec((bm, w), lambda i: (i, 0)),
                pl.BlockSpec((1, w), lambda i: (0, 0))],
      out_specs=pl.BlockSpec((bm, w), lambda i: (i, 0)),
      out_shape=jax.ShapeDtypeStruct((n, w), jnp.float32),
  )(a, cnt, z, bl)


def kernel(x, edge_index, Wl1, bl1, Wr1, g1, be1, Wl2, bl2, Wr2, g2, be2,
           Wl3, bl3, Wr3):
  n = x.shape[0]
  e = edge_index.shape[1]
  bm = 1000
  n_acc = -(-(n + 1) // (_NS * 64)) * (_NS * 64)
  cpt = -(-e // (_NS * _CHUNK))
  cpt = -(-cpt // 24) * 24
  ep = _NS * cpt * _CHUNK

  src = edge_index[0]
  dst = edge_index[1]
  packed = jnp.concatenate([
      src + dst * 65536,
      jnp.full((ep - e,), n * 65536, jnp.int32),
      jnp.zeros((8 * _CHUNK,), jnp.int32),
  ]).reshape(-1, _CHUNK)

  def r2(v):
    return v.reshape(1, -1)

  def flat(y):
    return y.reshape(-1, y.shape[2])

  cnt = _sc_degree_count(packed, n_acc).reshape(_NC, n_acc, 16)
  y1, z1 = _project_split(x, Wl1.T, Wr1.T, bm, n_acc)
  a1 = _sc_segment_sum(flat(y1), packed, n_acc).reshape(_NC, n_acc, -1)
  y2, z2 = _combine_project_split(a1, cnt, z1, r2(bl1), r2(g1),
                                  r2(be1), Wl2.T, Wr2.T, bm, n_acc)
  a2 = _sc_segment_sum(flat(y2), packed, n_acc).reshape(_NC, n_acc, -1)
  y3, z3 = _combine_project_split(a2, cnt, z2, r2(bl2), r2(g2),
                                  r2(be2), Wl3.T, Wr3.T, bm, n_acc)
  a3 = _sc_segment_sum(flat(y3), packed, n_acc).reshape(_NC, n_acc, -1)
  return _final_combine(a3, cnt, z3, r2(bl3), bm)
